# Initial kernel scaffold; baseline (speedup 1.0000x reference)
import jax
import jax.numpy as jnp
from jax.experimental import pallas as pl


def kernel(x, gate_W, gate_b, W1, b1, ln_g, ln_b, W2, b2, res_scale):
    raise NotImplementedError("write your pallas kernel here")


# speedup vs baseline: 3.0197x; 3.0197x over previous
import functools
import math

import jax
import jax.numpy as jnp
from jax import lax
from jax.experimental import pallas as pl
from jax.experimental.pallas import tpu as pltpu
from jax.experimental.pallas import tpu_sc as plsc

S = 2048
D = 1024
DFF = 2048
E = 8
TM = 128
P = S + E * TM
NT = P // TM
LANES = 128
NEG = -3e38


def _gate_body(x_ref, gwt_ref, bias_ref, e_ref):
    logits = lax.dot_general(x_ref[...], gwt_ref[...],
                             (((1,), (0,)), ((), ())),
                             preferred_element_type=jnp.float32)
    logits = logits + bias_ref[...]
    lane = lax.broadcasted_iota(jnp.int32, logits.shape, 1)
    big = jnp.int32(2 ** 30)
    m1 = jnp.max(logits, axis=1, keepdims=True)
    i1 = jnp.min(jnp.where(logits == m1, lane, big), axis=1, keepdims=True)
    masked = jnp.where(lane == i1, NEG, logits)
    m2 = jnp.max(masked, axis=1, keepdims=True)
    i2 = jnp.min(jnp.where(masked == m2, lane, big), axis=1, keepdims=True)
    e_ref[...] = jnp.maximum(i1, i2)


def _route_experts(xr, gate_W, gate_b):
    gwt = jnp.pad(gate_W.T, ((0, 0), (0, LANES - E)))
    bias = jnp.pad(gate_b, (0, LANES - E), constant_values=NEG)
    e2 = pl.pallas_call(
        _gate_body,
        out_shape=jax.ShapeDtypeStruct((S, 1), jnp.int32),
    )(xr, gwt, bias.reshape(1, LANES))
    return e2[:, 0]


@functools.lru_cache(maxsize=None)
def _sc_gather_fn(n_out, n_table):
    info = plsc.get_sparse_core_info()
    nw = info.num_cores * info.num_subcores
    b_per_w = n_out // nw
    mesh = plsc.VectorSubcoreMesh(core_axis_name="c", subcore_axis_name="s")

    @functools.partial(
        pl.kernel, mesh=mesh,
        out_type=jax.ShapeDtypeStruct((n_out, D), jnp.float32),
        scratch_types=[
            pltpu.VMEM((b_per_w,), jnp.int32),
            pltpu.VMEM((b_per_w, D), jnp.float32),
            pltpu.SemaphoreType.DMA,
        ],
    )
    def k(table_hbm, idx_hbm, out_hbm, idx_v, rows_v, sem):
        wid = lax.axis_index("s") * info.num_cores + lax.axis_index("c")
        base = wid * b_per_w
        pltpu.sync_copy(idx_hbm.at[pl.ds(base, b_per_w)], idx_v)
        pltpu.async_copy(table_hbm.at[idx_v], rows_v, sem).wait()
        pltpu.sync_copy(rows_v, out_hbm.at[pl.ds(base, b_per_w)])

    return k


def _gather_rows(table, idx):
    return _sc_gather_fn(idx.shape[0], table.shape[0])(table, idx)


def _ffn_body(te_ref, x_ref, w1_ref, b1_ref, g_ref, bb_ref, w2_ref, b2_ref,
              rs_ref, o_ref):
    x = x_ref[...]
    h = lax.dot_general(x, w1_ref[0], (((1,), (1,)), ((), ())),
                        preferred_element_type=jnp.float32)
    h = h + b1_ref[0]
    mu = jnp.mean(h, axis=1, keepdims=True)
    var = jnp.mean((h - mu) ** 2, axis=1, keepdims=True)
    h = (h - mu) / jnp.sqrt(var + 1e-5) * g_ref[0] + bb_ref[0]
    h = 0.5 * h * (1.0 + lax.erf(h * (1.0 / math.sqrt(2.0))))
    y = lax.dot_general(h, w2_ref[0], (((1,), (1,)), ((), ())),
                        preferred_element_type=jnp.float32)
    y = y + b2_ref[0]
    o_ref[...] = y * rs_ref[0] + x


def _grouped_ffn(xs, tile_e, W1, b1, ln_g, ln_b, W2, b2, res_scale):
    grid_spec = pltpu.PrefetchScalarGridSpec(
        num_scalar_prefetch=1,
        grid=(NT,),
        in_specs=[
            pl.BlockSpec((TM, D), lambda i, te: (i, 0)),
            pl.BlockSpec((1, DFF, D), lambda i, te: (te[i], 0, 0)),
            pl.BlockSpec((1, 1, DFF), lambda i, te: (te[i], 0, 0)),
            pl.BlockSpec((1, 1, DFF), lambda i, te: (te[i], 0, 0)),
            pl.BlockSpec((1, 1, DFF), lambda i, te: (te[i], 0, 0)),
            pl.BlockSpec((1, D, DFF), lambda i, te: (te[i], 0, 0)),
            pl.BlockSpec((1, 1, D), lambda i, te: (te[i], 0, 0)),
            pl.BlockSpec((1, 1, 1), lambda i, te: (te[i], 0, 0)),
        ],
        out_specs=pl.BlockSpec((TM, D), lambda i, te: (i, 0)),
    )
    return pl.pallas_call(
        _ffn_body,
        grid_spec=grid_spec,
        out_shape=jax.ShapeDtypeStruct((P, D), jnp.float32),
    )(tile_e, xs, W1, b1.reshape(E, 1, DFF), ln_g.reshape(E, 1, DFF),
      ln_b.reshape(E, 1, DFF), W2, b2.reshape(E, 1, D),
      res_scale.reshape(E, 1, 1))


def kernel(x, gate_W, gate_b, W1, b1, ln_g, ln_b, W2, b2, res_scale):
    xr = x.reshape(S, D)
    e_star = _route_experts(xr, gate_W, gate_b)

    oh = (e_star[:, None] == jnp.arange(E, dtype=jnp.int32)[None, :])
    cum = jnp.cumsum(oh.astype(jnp.int32), axis=0)
    counts = cum[-1]
    tiles_pe = (counts + TM - 1) // TM
    tile_cum = jnp.cumsum(tiles_pe)
    tile_start = (tile_cum - tiles_pe) * TM
    rank = jnp.sum(jnp.where(oh, cum, 0), axis=1) - 1
    pos = (tile_start[e_star] + rank).astype(jnp.int32)
    src = jnp.zeros((P,), jnp.int32).at[pos].set(
        jnp.arange(S, dtype=jnp.int32))

    tidx = jnp.arange(NT, dtype=jnp.int32)
    te_raw = jnp.searchsorted(tile_cum, tidx, side="right").astype(jnp.int32)
    n_valid = tile_cum[E - 1]
    last_e = te_raw[jnp.maximum(n_valid - 1, 0)]
    tile_e = jnp.where(tidx < n_valid, te_raw, last_e)

    xs = _gather_rows(xr, src)
    ys = _grouped_ffn(xs, tile_e, W1, b1, ln_g, ln_b, W2, b2, res_scale)
    out = _gather_rows(ys, pos)
    return out.reshape(x.shape)
